# baseline (speedup 1.0000x reference)
import functools
import math

import jax
import jax.numpy as jnp
import numpy as np
from jax import lax
from jax.experimental import pallas as pl
from jax.experimental.pallas import tpu as pltpu
from jax.experimental.pallas import tpu_sc as plsc

NUM_CORES = 2
NUM_SUBCORES = 16
NW = NUM_CORES * NUM_SUBCORES
LANES = 16

VOCAB = 32000
EMB_DIM = 2048
BATCH = 4
SEQ = 2048
PPW = SEQ // NW
K = 8
NGRP = PPW // K
NCH = NGRP * BATCH
GROUPS = EMB_DIM // LANES
NROWS = 4
NPE = 2


@functools.lru_cache(maxsize=None)
def _pos_encoding(seq_len, d_model):
    position = np.arange(seq_len, dtype=np.float32)[:, None]
    div_term = np.exp(
        np.arange(0, d_model, 2, dtype=np.float32)
        * (-math.log(10000.0) / d_model))
    pe = np.zeros((seq_len, d_model), dtype=np.float32)
    pe[:, 0::2] = np.sin(position * div_term)
    pe[:, 1::2] = np.cos(position * div_term)
    return pe


def _sc_body(idx_hbm, pe_hbm, table_hbm, out_hbm, idx_v,
             r0, r1, r2, r3, p0, p1,
             g0s, g1s, g2s, g3s, p0s, p1s, o0s, o1s, o2s, o3s):
    rows = [r0, r1, r2, r3]
    pes = [p0, p1]
    gsems = [g0s, g1s, g2s, g3s]
    psems = [p0s, p1s]
    osems = [o0s, o1s, o2s, o3s]

    wid = lax.axis_index("s") * NUM_CORES + lax.axis_index("c")
    pos0 = wid * PPW

    for b in range(BATCH):
        pltpu.sync_copy(idx_hbm.at[b, pl.ds(pos0, PPW)],
                        idx_v.at[pl.ds(b * PPW, PPW)])

    def issue_g(c, rb):
        off = (c & 3) * PPW + lax.shift_right_logical(c, 2) * K
        pltpu.async_copy(table_hbm.at[idx_v.at[pl.ds(off, K)]],
                         rows[rb], gsems[rb])

    def issue_p(q, pb):
        pltpu.async_copy(pe_hbm.at[pl.ds(pos0 + q * K, K)],
                         pes[pb], psems[pb])

    def issue_w(c, rb):
        b = c & 3
        q = lax.shift_right_logical(c, 2)
        pltpu.async_copy(rows[rb],
                         out_hbm.at[b, pl.ds(pos0 + q * K, K)],
                         osems[rb])

    def wait_g(rb):
        pltpu.make_async_copy(table_hbm.at[pl.ds(0, K)],
                              rows[rb], gsems[rb]).wait()

    def wait_p(pb):
        pltpu.make_async_copy(pe_hbm.at[pl.ds(0, K)],
                              pes[pb], psems[pb]).wait()

    def wait_w(rb):
        pltpu.make_async_copy(rows[rb], out_hbm.at[0, pl.ds(0, K)],
                              osems[rb]).wait()

    def process_pair(c0, rb0, pb, wait_pe, prefetch, drain_prev):
        wait_g(rb0)
        wait_g(rb0 + 1)
        if wait_pe:
            wait_p(pb)
        row0 = rows[rb0]
        row1 = rows[rb0 + 1]
        pe_ref = pes[pb]

        UNROLL = 16

        def lane_add(i, carry):
            r = lax.shift_right_logical(i, 3)
            g0 = (i & 7) * UNROLL * LANES
            for u in range(UNROLL):
                sl = pl.ds(g0 + u * LANES, LANES)
                pv = pe_ref[r, sl]
                row0[r, sl] = row0[r, sl] + pv
                row1[r, sl] = row1[r, sl] + pv
            return carry

        lax.fori_loop(0, K * GROUPS // UNROLL, lane_add, None)

        issue_w(c0, rb0)
        issue_w(c0 + 1, rb0 + 1)
        if prefetch:
            rb2 = (rb0 + 2) % NROWS
            if drain_prev:
                wait_w(rb2)
                wait_w(rb2 + 1)
            issue_g(c0 + 2, rb2)
            issue_g(c0 + 3, rb2 + 1)

    issue_p(0, 0)
    issue_g(0, 0)
    issue_g(1, 1)
    issue_p(1, 1)
    process_pair(0, 0, 0, wait_pe=True, prefetch=True, drain_prev=False)
    process_pair(2, 2, 0, wait_pe=False, prefetch=True, drain_prev=True)

    def pair(i, _):
        c0 = 4 + 8 * i
        issue_p(2 * i + 2, 0)
        process_pair(c0 + 0, 0, 1, wait_pe=True, prefetch=True,
                     drain_prev=True)
        process_pair(c0 + 2, 2, 1, wait_pe=False, prefetch=True,
                     drain_prev=True)
        issue_p(2 * i + 3, 1)
        process_pair(c0 + 4, 0, 0, wait_pe=True, prefetch=True,
                     drain_prev=True)
        process_pair(c0 + 6, 2, 0, wait_pe=False, prefetch=True,
                     drain_prev=True)
        return _

    lax.fori_loop(0, (NGRP - 2) // 2, pair, None)

    process_pair(NCH - 4, 0, 1, wait_pe=True, prefetch=True,
                 drain_prev=True)
    process_pair(NCH - 2, 2, 1, wait_pe=False, prefetch=False,
                 drain_prev=False)
    for rb in range(NROWS):
        wait_w(rb)


@jax.jit
def _run(idx, pe, table):
    kfn = pl.kernel(
        _sc_body,
        out_type=jax.ShapeDtypeStruct((BATCH, SEQ, EMB_DIM), jnp.float32),
        mesh=plsc.VectorSubcoreMesh(
            core_axis_name="c", subcore_axis_name="s",
            num_cores=NUM_CORES, num_subcores=NUM_SUBCORES),
        scratch_types=(
            [pltpu.VMEM((BATCH * PPW,), jnp.int32)]
            + [pltpu.VMEM((K, EMB_DIM), jnp.float32)] * NROWS
            + [pltpu.VMEM((K, EMB_DIM), jnp.float32)] * NPE
            + [pltpu.SemaphoreType.DMA] * (NROWS + NPE + NROWS)
        ),
    )
    return kfn(idx, pe, table)


def kernel(x, emb_table):
    idx = x.astype(jnp.int32)
    pe = _pos_encoding(SEQ, EMB_DIM)
    return _run(idx, pe, emb_table)


# speedup vs baseline: 1.1105x; 1.1105x over previous
import functools
import math

import jax
import jax.numpy as jnp
import numpy as np
from jax import lax
from jax.experimental import pallas as pl
from jax.experimental.pallas import tpu as pltpu
from jax.experimental.pallas import tpu_sc as plsc

NUM_CORES = 2
NUM_SUBCORES = 16
NW = NUM_CORES * NUM_SUBCORES
LANES = 16

VOCAB = 32000
EMB_DIM = 2048
BATCH = 4
SEQ = 2048
PPW = SEQ // NW
K = 8
NGRP = PPW // K
NCH = NGRP * BATCH
GROUPS = EMB_DIM // LANES
NROWS = 4
NPE = 2


@functools.lru_cache(maxsize=None)
def _pos_encoding(seq_len, d_model):
    position = np.arange(seq_len, dtype=np.float32)[:, None]
    div_term = np.exp(
        np.arange(0, d_model, 2, dtype=np.float32)
        * (-math.log(10000.0) / d_model))
    pe = np.zeros((seq_len, d_model), dtype=np.float32)
    pe[:, 0::2] = np.sin(position * div_term)
    pe[:, 1::2] = np.cos(position * div_term)
    return pe


def _sc_body(idx_hbm, pe_hbm, table_hbm, out_hbm, idx_v,
             r0, r1, r2, r3, p0, p1,
             g0s, g1s, g2s, g3s, p0s, p1s, o0s, o1s, o2s, o3s):
    rows = [r0, r1, r2, r3]
    pes = [p0, p1]
    gsems = [g0s, g1s, g2s, g3s]
    psems = [p0s, p1s]
    osems = [o0s, o1s, o2s, o3s]

    wid = lax.axis_index("s") * NUM_CORES + lax.axis_index("c")
    pos0 = wid * PPW

    for b in range(BATCH):
        pltpu.sync_copy(idx_hbm.at[b, pl.ds(pos0, PPW)],
                        idx_v.at[pl.ds(b * PPW, PPW)])

    def issue_g(c, rb):
        off = (c & 3) * PPW + lax.shift_right_logical(c, 2) * K
        pltpu.async_copy(table_hbm.at[idx_v.at[pl.ds(off, K)]],
                         rows[rb], gsems[rb])

    def issue_p(q, pb):
        pltpu.async_copy(pe_hbm.at[pl.ds(pos0 + q * K, K)],
                         pes[pb], psems[pb])

    def issue_w(c, rb):
        b = c & 3
        q = lax.shift_right_logical(c, 2)
        pltpu.async_copy(rows[rb],
                         out_hbm.at[b, pl.ds(pos0 + q * K, K)],
                         osems[rb])

    def wait_g(rb):
        pltpu.make_async_copy(table_hbm.at[pl.ds(0, K)],
                              rows[rb], gsems[rb]).wait()

    def wait_p(pb):
        pltpu.make_async_copy(pe_hbm.at[pl.ds(0, K)],
                              pes[pb], psems[pb]).wait()

    def wait_w(rb):
        pltpu.make_async_copy(rows[rb], out_hbm.at[0, pl.ds(0, K)],
                              osems[rb]).wait()

    def process(c, rb, pb, wait_pe, prefetch, drain_prev):
        wait_g(rb)
        if wait_pe:
            wait_p(pb)
        if prefetch:
            rb2 = (rb + 2) % NROWS
            if drain_prev:
                wait_w(rb2)
            issue_g(c + 2, rb2)
        row_ref = rows[rb]
        pe_ref = pes[pb]

        UNROLL = 16

        def lane_add(i, carry):
            r = lax.shift_right_logical(i, 3)
            g0 = (i & 7) * UNROLL * LANES
            for u in range(UNROLL):
                sl = pl.ds(g0 + u * LANES, LANES)
                row_ref[r, sl] = row_ref[r, sl] + pe_ref[r, sl]
            return carry

        lax.fori_loop(0, K * GROUPS // UNROLL, lane_add, None)

        issue_w(c, rb)

    issue_p(0, 0)
    issue_g(0, 0)
    issue_g(1, 1)
    issue_p(1, 1)
    process(0, 0, 0, wait_pe=True, prefetch=True, drain_prev=False)
    process(1, 1, 0, wait_pe=False, prefetch=True, drain_prev=False)
    process(2, 2, 0, wait_pe=False, prefetch=True, drain_prev=True)
    process(3, 3, 0, wait_pe=False, prefetch=True, drain_prev=True)

    def pair(i, _):
        c0 = 4 + 8 * i
        issue_p(2 * i + 2, 0)
        process(c0 + 0, 0, 1, wait_pe=True, prefetch=True, drain_prev=True)
        process(c0 + 1, 1, 1, wait_pe=False, prefetch=True, drain_prev=True)
        process(c0 + 2, 2, 1, wait_pe=False, prefetch=True, drain_prev=True)
        process(c0 + 3, 3, 1, wait_pe=False, prefetch=True, drain_prev=True)
        issue_p(2 * i + 3, 1)
        process(c0 + 4, 0, 0, wait_pe=True, prefetch=True, drain_prev=True)
        process(c0 + 5, 1, 0, wait_pe=False, prefetch=True, drain_prev=True)
        process(c0 + 6, 2, 0, wait_pe=False, prefetch=True, drain_prev=True)
        process(c0 + 7, 3, 0, wait_pe=False, prefetch=True, drain_prev=True)
        return _

    lax.fori_loop(0, (NGRP - 2) // 2, pair, None)

    process(NCH - 4, 0, 1, wait_pe=True, prefetch=True, drain_prev=True)
    process(NCH - 3, 1, 1, wait_pe=False, prefetch=True, drain_prev=True)
    process(NCH - 2, 2, 1, wait_pe=False, prefetch=False, drain_prev=False)
    process(NCH - 1, 3, 1, wait_pe=False, prefetch=False, drain_prev=False)
    for rb in range(NROWS):
        wait_w(rb)


@jax.jit
def _run(idx, pe, table):
    kfn = pl.kernel(
        _sc_body,
        out_type=jax.ShapeDtypeStruct((BATCH, SEQ, EMB_DIM), jnp.float32),
        mesh=plsc.VectorSubcoreMesh(
            core_axis_name="c", subcore_axis_name="s",
            num_cores=NUM_CORES, num_subcores=NUM_SUBCORES),
        scratch_types=(
            [pltpu.VMEM((BATCH * PPW,), jnp.int32)]
            + [pltpu.VMEM((K, EMB_DIM), jnp.float32)] * NROWS
            + [pltpu.VMEM((K, EMB_DIM), jnp.float32)] * NPE
            + [pltpu.SemaphoreType.DMA] * (NROWS + NPE + NROWS)
        ),
    )
    return kfn(idx, pe, table)


def kernel(x, emb_table):
    idx = x.astype(jnp.int32)
    pe = _pos_encoding(SEQ, EMB_DIM)
    return _run(idx, pe, emb_table)
